# Initial kernel scaffold; baseline (speedup 1.0000x reference)
import jax
import jax.numpy as jnp
from jax.experimental import pallas as pl


def kernel(qk_logits_btmd, v_logits_btmd, v_emb_mk_d):
    raise NotImplementedError("write your pallas kernel here")


# speedup vs baseline: 53.4908x; 53.4908x over previous
import functools

import jax
import jax.numpy as jnp
from jax import lax
from jax.experimental import pallas as pl
from jax.experimental.pallas import tpu as pltpu
from jax.experimental.pallas import tpu_sc as plsc


def _pre_body(Dqk, Dv, N, T, qk_ref, v_ref, q_out, vid_out, e_out, rank_out):
    i32 = jnp.int32
    best = qk_ref[0]
    bidx = jnp.zeros((N, T), i32)
    for d in range(1, Dqk):
        x = qk_ref[d]
        gt = x > best
        best = jnp.where(gt, x, best)
        bidx = jnp.where(gt, d, bidx)
    q = bidx
    q_out[...] = q

    vbest = v_ref[0]
    vidx = jnp.zeros((N, T), i32)
    for d in range(1, Dv):
        x = v_ref[d]
        gt = x > vbest
        vbest = jnp.where(gt, x, vbest)
        vidx = jnp.where(gt, d, vidx)
    vid_out[...] = vidx

    qnext = jnp.concatenate([q[:, 1:], jnp.full((N, 1), -1, i32)], axis=1)
    is_end = q != qnext
    iota_t = lax.broadcasted_iota(i32, (N, T), 1)
    e = jnp.where(is_end, iota_t, T)
    sh = 1
    while sh < T:
        shifted = jnp.concatenate(
            [e[:, sh:], jnp.full((N, sh), T, i32)], axis=1)
        e = jnp.minimum(e, shifted)
        sh *= 2
    e_out[...] = e

    rank = jnp.zeros((N, T), i32)
    for c in range(Dqk):
        cnt = (q == c).astype(i32)
        sh = 1
        while sh < T:
            shifted = jnp.concatenate(
                [jnp.zeros((N, sh), i32), cnt[:, :T - sh]], axis=1)
            cnt = cnt + shifted
            sh *= 2
        rank = rank + jnp.where(q == c, cnt - 1, 0)
    rank_out[...] = rank


def _sload(ref, i):
    return ref[pl.ds(i, 16)][0]


def _dp_body(N, T, Dv, q_hbm, e_hbm, rank_hbm, vid_hbm, vsel_hbm,
             q_v, e_v, rank_v, bucket_v, dbuf_v, tau_v, vid_v):
    i32 = jnp.int32
    cid = lax.axis_index("c")
    sid = lax.axis_index("s")
    wid = sid * 2 + cid

    @pl.when(wid < N)
    def _():
        row = wid
        pltpu.sync_copy(q_hbm.at[row], q_v.at[pl.ds(0, T)])
        pltpu.sync_copy(e_hbm.at[row], e_v.at[pl.ds(0, T)])
        pltpu.sync_copy(rank_hbm.at[row], rank_v.at[pl.ds(0, T)])
        iota = lax.iota(i32, 16)
        zeros16 = jnp.zeros((16,), i32)

        def zero_body(k, carry):
            dbuf_v[pl.ds(k * 16, 16)] = zeros16
            return carry
        lax.fori_loop(0, (T + 16) // 16, zero_body, 0)

        def bk_body(k, carry):
            tv = k * 16 + iota
            qv = q_v[pl.ds(k * 16, 16)]
            rv = rank_v[pl.ds(k * 16, 16)]
            plsc.store_scatter(bucket_v, [qv * T + rv], tv)
            return carry
        lax.fori_loop(0, T // 16, bk_body, 0)

        def run_body(i0):
            c = _sload(q_v, i0)
            ecur = _sload(e_v, i0)
            r0 = _sload(rank_v, i0)
            L = ecur - i0 + 1

            def no_prior():
                def fill(k, carry):
                    t = k * 16 + iota
                    m = t < L
                    plsc.store_scatter(
                        tau_v, [i0 + t], jnp.full((16,), -1, i32), mask=m)
                    return carry
                lax.fori_loop(0, (L + 15) // 16, fill, 0)
                return 0

            def with_prior():
                base = c * T

                def gmax(k, best):
                    lane = k * 16 + iota
                    m = lane < r0
                    j = plsc.load_gather(bucket_v, [base + lane], mask=m)
                    j = jnp.where(m, j, 0)
                    s = plsc.load_gather(dbuf_v, [j], mask=m) + 1
                    key = jnp.where(m, s * 4096 + j, -1)
                    return jnp.maximum(best, jnp.max(key))
                best = lax.fori_loop(0, (r0 + 15) // 16, gmax, i32(-1))
                w0 = jnp.bitwise_and(best, 4095)
                s0 = lax.shift_right_arithmetic(best, 12)
                e0 = _sload(e_v, w0)

                def wtau(k, carry):
                    t = k * 16 + iota
                    m = t < L
                    plsc.store_scatter(
                        tau_v, [i0 + t], jnp.minimum(w0 + t, e0), mask=m)
                    return carry
                lax.fori_loop(0, (L + 15) // 16, wtau, 0)

                wlen = jnp.minimum(L, e0 - w0 + 1)

                def wd(k, carry):
                    t = k * 16 + iota
                    m = t < wlen
                    plsc.store_scatter(dbuf_v, [w0 + 1 + t], s0 + t, mask=m)
                    return carry
                lax.fori_loop(0, (wlen + 15) // 16, wd, 0)
                return 0

            lax.cond(r0 == 0, no_prior, with_prior)
            return ecur + 1

        lax.while_loop(lambda i0: i0 < T, run_body, i32(0))

        pltpu.sync_copy(vid_hbm.at[row], vid_v.at[pl.ds(0, T)])

        def sel_body(k, carry):
            tv = tau_v[pl.ds(k * 16, 16)]
            tc = jnp.clip(tv, 0, T - 1)
            vs = plsc.load_gather(vid_v, [tc])
            tau_v[pl.ds(k * 16, 16)] = jnp.where(tv >= 0, vs, Dv)
            return carry
        lax.fori_loop(0, T // 16, sel_body, 0)
        pltpu.sync_copy(tau_v, vsel_hbm.at[row])


def _emb_body(M, Dv, T, d_out, vsel_ref, vemb_ref, out_ref):
    i32 = jnp.int32
    iota_dv_t = lax.broadcasted_iota(i32, (Dv, T), 0)
    for m in range(M):
        vs = vsel_ref[0, m, :]
        ohT = (vs[None, :] == iota_dv_t).astype(jnp.float32)
        res = lax.dot_general(
            ohT, vemb_ref[m], (((0,), (0,)), ((), ())),
            precision=lax.Precision.HIGHEST,
            preferred_element_type=jnp.float32)
        out_ref[0, :, m * d_out:(m + 1) * d_out] = res


def kernel(qk_logits_btmd, v_logits_btmd, v_emb_mk_d):
    B, T, M, Dqk = qk_logits_btmd.shape
    Dv = v_logits_btmd.shape[-1]
    d_out = v_emb_mk_d.shape[-1]
    N = B * M
    i32 = jnp.int32

    qk_dnt = qk_logits_btmd.transpose(3, 0, 2, 1).reshape(Dqk, N, T)
    v_dnt = v_logits_btmd.transpose(3, 0, 2, 1).reshape(Dv, N, T)

    q_nt, vid_nt, e_nt, rank_nt = pl.pallas_call(
        functools.partial(_pre_body, Dqk, Dv, N, T),
        out_shape=[jax.ShapeDtypeStruct((N, T), i32)] * 4,
    )(qk_dnt, v_dnt)

    mesh = plsc.VectorSubcoreMesh(core_axis_name="c", subcore_axis_name="s")
    sc_params = pltpu.CompilerParams(needs_layout_passes=False)
    vsel_nt = pl.kernel(
        functools.partial(_dp_body, N, T, Dv),
        out_type=jax.ShapeDtypeStruct((N, T), i32),
        mesh=mesh,
        scratch_types=[
            pltpu.VMEM((T + 16,), i32),
            pltpu.VMEM((T + 16,), i32),
            pltpu.VMEM((T + 16,), i32),
            pltpu.VMEM((Dqk * T,), i32),
            pltpu.VMEM((T + 16,), i32),
            pltpu.VMEM((T,), i32),
            pltpu.VMEM((T,), i32),
        ],
        compiler_params=sc_params,
    )(q_nt, e_nt, rank_nt, vid_nt)

    vsel_bmt = vsel_nt.reshape(B, M, T)
    out_flat = pl.pallas_call(
        functools.partial(_emb_body, M, Dv, T, d_out),
        grid=(B,),
        in_specs=[
            pl.BlockSpec((1, M, T), lambda b: (b, 0, 0)),
            pl.BlockSpec((M, Dv, d_out), lambda b: (0, 0, 0)),
        ],
        out_specs=pl.BlockSpec((1, T, M * d_out), lambda b: (b, 0, 0)),
        out_shape=jax.ShapeDtypeStruct((B, T, M * d_out), jnp.float32),
    )(vsel_bmt, v_emb_mk_d)

    return out_flat.reshape(B, T, M, d_out)


# speedup vs baseline: 62.3105x; 1.1649x over previous
import functools

import jax
import jax.numpy as jnp
from jax import lax
from jax.experimental import pallas as pl
from jax.experimental.pallas import tpu as pltpu
from jax.experimental.pallas import tpu_sc as plsc


def _pre_body(Dqk, Dv, N, T, qk_ref, v_ref, q_out, vid_out, er_out):
    i32 = jnp.int32
    best = qk_ref[0]
    bidx = jnp.zeros((N, T), i32)
    for d in range(1, Dqk):
        x = qk_ref[d]
        gt = x > best
        best = jnp.where(gt, x, best)
        bidx = jnp.where(gt, d, bidx)
    q = bidx
    q_out[...] = q

    vbest = v_ref[0]
    vidx = jnp.zeros((N, T), i32)
    for d in range(1, Dv):
        x = v_ref[d]
        gt = x > vbest
        vbest = jnp.where(gt, x, vbest)
        vidx = jnp.where(gt, d, vidx)
    vid_out[...] = vidx

    qnext = jnp.concatenate([q[:, 1:], jnp.full((N, 1), -1, i32)], axis=1)
    is_end = q != qnext
    iota_t = lax.broadcasted_iota(i32, (N, T), 1)
    e = jnp.where(is_end, iota_t, T)
    sh = 1
    while sh < T:
        shifted = jnp.concatenate(
            [e[:, sh:], jnp.full((N, sh), T, i32)], axis=1)
        e = jnp.minimum(e, shifted)
        sh *= 2

    rank = jnp.zeros((N, T), i32)
    for c in range(Dqk):
        cnt = (q == c).astype(i32)
        sh = 1
        while sh < T:
            shifted = jnp.concatenate(
                [jnp.zeros((N, sh), i32), cnt[:, :T - sh]], axis=1)
            cnt = cnt + shifted
            sh *= 2
        rank = rank + jnp.where(q == c, cnt - 1, 0)
    er_out[...] = e * T + rank


def _sload(ref, i):
    return ref[pl.ds(i, 16)][0]


def _dp_body(N, T, Dv, q_hbm, er_hbm, vid_hbm, vsel_hbm,
             q_v, er_v, key_v, tau_v, vid_v):
    i32 = jnp.int32
    TSH = int(T).bit_length() - 1
    cid = lax.axis_index("c")
    sid = lax.axis_index("s")
    wid = sid * 2 + cid

    @pl.when(wid < N)
    def _():
        row = wid
        pltpu.sync_copy(q_hbm.at[row], q_v.at[pl.ds(0, T)])
        pltpu.sync_copy(er_hbm.at[row], er_v.at[pl.ds(0, T)])
        pltpu.sync_copy(vid_hbm.at[row], vid_v.at[pl.ds(0, T)])
        iota = lax.iota(i32, 16)
        neg1 = jnp.full((16,), -1, i32)

        def bk_body(k, carry):
            tv = k * 16 + iota
            qv = q_v[pl.ds(k * 16, 16)]
            rv = jnp.bitwise_and(er_v[pl.ds(k * 16, 16)], T - 1)
            plsc.store_scatter(key_v, [qv * T + rv], 4096 + tv)
            return carry
        lax.fori_loop(0, T // 16, bk_body, 0)

        def run_body(i0):
            c = _sload(q_v, i0)
            er0 = _sload(er_v, i0)
            ecur = lax.shift_right_logical(er0, TSH)
            r0 = jnp.bitwise_and(er0, T - 1)
            L = ecur - i0 + 1
            nchunk = (L + 15) // 16
            base = c * T

            nfull = lax.shift_right_logical(r0, 4)

            def gchunk(k, acc):
                return jnp.maximum(acc, key_v[pl.ds(base + k * 16, 16)])
            acc = lax.fori_loop(0, nfull, gchunk, neg1)
            tailv = key_v[pl.ds(base + nfull * 16, 16)]
            rem = jnp.bitwise_and(r0, 15)
            best = jnp.max(jnp.maximum(acc, jnp.where(iota < rem, tailv, -1)))

            def no_prior():
                def fill(k, carry):
                    tau_v[pl.ds(i0 + k * 16, 16)] = neg1
                    return carry
                lax.fori_loop(0, nchunk, fill, 0)
                return 0

            def with_prior():
                w0 = jnp.bitwise_and(best, 4095)
                s0 = lax.shift_right_arithmetic(best, 12)
                erw = _sload(er_v, w0)
                e0 = lax.shift_right_logical(erw, TSH)
                r0w = jnp.bitwise_and(erw, T - 1)

                def wtau(k, carry):
                    tau_v[pl.ds(i0 + k * 16, 16)] = jnp.minimum(
                        w0 + k * 16 + iota, e0)
                    return carry
                lax.fori_loop(0, nchunk, wtau, 0)

                wlen = jnp.minimum(L, e0 - w0 + 1)
                ns = jnp.minimum(wlen, e0 - w0)
                kbase = base + r0w + 1
                vbase = (s0 + 1) * 4096 + w0 + 1

                def wkey(k, carry):
                    t = k * 16 + iota
                    plsc.store_scatter(
                        key_v, [kbase + t], vbase + t * 4097, mask=t < ns)
                    return carry
                lax.fori_loop(0, (ns + 15) // 16, wkey, 0)

                @pl.when((wlen > e0 - w0) & (e0 + 1 < T))
                def _():
                    cb = _sload(q_v, e0 + 1)
                    rb = jnp.bitwise_and(_sload(er_v, e0 + 1), T - 1)
                    plsc.store_scatter(
                        key_v, [cb * T + rb + iota],
                        jnp.full((16,), (s0 + wlen) * 4096 + e0 + 1, i32),
                        mask=iota < 1)
                return 0

            lax.cond(r0 == 0, no_prior, with_prior)
            return ecur + 1

        lax.while_loop(lambda i0: i0 < T, run_body, i32(0))

        def sel_body(k, carry):
            tv = tau_v[pl.ds(k * 16, 16)]
            tc = jnp.clip(tv, 0, T - 1)
            vs = plsc.load_gather(vid_v, [tc])
            tau_v[pl.ds(k * 16, 16)] = jnp.where(tv >= 0, vs, Dv)
            return carry
        lax.fori_loop(0, T // 16, sel_body, 0)
        pltpu.sync_copy(tau_v.at[pl.ds(0, T)], vsel_hbm.at[row])


def _emb_body(M, Dv, T, d_out, vsel_ref, vemb_ref, out_ref):
    i32 = jnp.int32
    iota_dv_t = lax.broadcasted_iota(i32, (Dv, T), 0)
    for m in range(M):
        vs = vsel_ref[0, m, :]
        ohT = (vs[None, :] == iota_dv_t).astype(jnp.float32)
        res = lax.dot_general(
            ohT, vemb_ref[m], (((0,), (0,)), ((), ())),
            precision=lax.Precision.HIGHEST,
            preferred_element_type=jnp.float32)
        out_ref[0, :, m * d_out:(m + 1) * d_out] = res


def kernel(qk_logits_btmd, v_logits_btmd, v_emb_mk_d):
    B, T, M, Dqk = qk_logits_btmd.shape
    Dv = v_logits_btmd.shape[-1]
    d_out = v_emb_mk_d.shape[-1]
    N = B * M
    i32 = jnp.int32

    qk_dnt = qk_logits_btmd.transpose(3, 0, 2, 1).reshape(Dqk, N, T)
    v_dnt = v_logits_btmd.transpose(3, 0, 2, 1).reshape(Dv, N, T)

    q_nt, vid_nt, er_nt = pl.pallas_call(
        functools.partial(_pre_body, Dqk, Dv, N, T),
        out_shape=[jax.ShapeDtypeStruct((N, T), i32)] * 3,
    )(qk_dnt, v_dnt)

    mesh = plsc.VectorSubcoreMesh(core_axis_name="c", subcore_axis_name="s")
    sc_params = pltpu.CompilerParams(needs_layout_passes=False)
    vsel_nt = pl.kernel(
        functools.partial(_dp_body, N, T, Dv),
        out_type=jax.ShapeDtypeStruct((N, T), i32),
        mesh=mesh,
        scratch_types=[
            pltpu.VMEM((T + 16,), i32),
            pltpu.VMEM((T + 16,), i32),
            pltpu.VMEM((Dqk * T + 16,), i32),
            pltpu.VMEM((T + 16,), i32),
            pltpu.VMEM((T,), i32),
        ],
        compiler_params=sc_params,
    )(q_nt, er_nt, vid_nt)

    vsel_bmt = vsel_nt.reshape(B, M, T)
    out_flat = pl.pallas_call(
        functools.partial(_emb_body, M, Dv, T, d_out),
        grid=(B,),
        in_specs=[
            pl.BlockSpec((1, M, T), lambda b: (b, 0, 0)),
            pl.BlockSpec((M, Dv, d_out), lambda b: (0, 0, 0)),
        ],
        out_specs=pl.BlockSpec((1, T, M * d_out), lambda b: (b, 0, 0)),
        out_shape=jax.ShapeDtypeStruct((B, T, M * d_out), jnp.float32),
    )(vsel_bmt, v_emb_mk_d)

    return out_flat.reshape(B, T, M, d_out)


# speedup vs baseline: 81.7844x; 1.3125x over previous
import functools

import jax
import jax.numpy as jnp
from jax import lax
from jax.experimental import pallas as pl
from jax.experimental.pallas import tpu as pltpu
from jax.experimental.pallas import tpu_sc as plsc


def _pre_body(Dqk, Dv, N, T, qk_ref, v_ref, q_out, vid_out, e_out):
    i32 = jnp.int32
    best = qk_ref[0]
    bidx = jnp.zeros((N, T), i32)
    for d in range(1, Dqk):
        x = qk_ref[d]
        gt = x > best
        best = jnp.where(gt, x, best)
        bidx = jnp.where(gt, d, bidx)
    q = bidx
    q_out[...] = q

    vbest = v_ref[0]
    vidx = jnp.zeros((N, T), i32)
    for d in range(1, Dv):
        x = v_ref[d]
        gt = x > vbest
        vbest = jnp.where(gt, x, vbest)
        vidx = jnp.where(gt, d, vidx)
    vid_out[...] = vidx

    qnext = jnp.concatenate([q[:, 1:], jnp.full((N, 1), -1, i32)], axis=1)
    is_end = q != qnext
    iota_t = lax.broadcasted_iota(i32, (N, T), 1)
    e = jnp.where(is_end, iota_t, T)
    sh = 1
    while sh < T:
        shifted = jnp.concatenate(
            [e[:, sh:], jnp.full((N, sh), T, i32)], axis=1)
        e = jnp.minimum(e, shifted)
        sh *= 2
    e_out[...] = e


def _sload(ref, i):
    return ref[pl.ds(i, 16)][0]


def _dp_body(N, T, Dv, q_hbm, e_hbm, vid_hbm, vsel_hbm,
             q_v, e_v, tau_v, vid_v, bk_v):
    i32 = jnp.int32
    cid = lax.axis_index("c")
    sid = lax.axis_index("s")
    wid = sid * 2 + cid

    @pl.when(wid < N)
    def _():
        row = wid
        pltpu.sync_copy(q_hbm.at[row], q_v.at[pl.ds(0, T)])
        pltpu.sync_copy(e_hbm.at[row], e_v.at[pl.ds(0, T)])
        pltpu.sync_copy(vid_hbm.at[row], vid_v.at[pl.ds(0, T)])
        iota = lax.iota(i32, 16)
        neg1 = jnp.full((16,), -1, i32)
        zeros16 = jnp.zeros((16,), i32)

        @plsc.parallel_loop(0, 17)
        def _init(k):
            bk_v[pl.ds(k * 16, 16)] = neg1

        def run_body(i0):
            c = _sload(q_v, i0)
            ecur = _sload(e_v, i0)
            best = _sload(bk_v, c * 16)
            L = ecur - i0 + 1
            nchunk = (L + 15) // 16

            def no_prior():
                def fill(k, carry):
                    tau_v[pl.ds(i0 + k * 16, 16)] = neg1
                    return carry
                lax.fori_loop(0, nchunk, fill, 0)
                return 4096 + ecur

            def with_prior():
                w0 = jnp.bitwise_and(best, 4095)
                s0 = lax.shift_right_arithmetic(best, 12)
                e0 = _sload(e_v, w0)

                def wtau(k, carry):
                    tau_v[pl.ds(i0 + k * 16, 16)] = jnp.minimum(
                        w0 + k * 16 + iota, e0)
                    return carry
                lax.fori_loop(0, nchunk, wtau, 0)

                wlen = jnp.minimum(L, e0 - w0 + 1)
                ns = jnp.minimum(wlen, e0 - w0)

                @pl.when((wlen > e0 - w0) & (e0 + 1 < T))
                def _():
                    cb = _sload(q_v, e0 + 1)
                    old = _sload(bk_v, cb * 16)
                    nb = jnp.maximum(old, (s0 + wlen) * 4096 + e0 + 1)
                    bk_v[pl.ds(cb * 16, 16)] = zeros16 + nb

                return jnp.where(ns > 0, (s0 + ns) * 4096 + w0 + ns,
                                 4096 + ecur)

            val_c = lax.cond(best < 0, no_prior, with_prior)
            bk_v[pl.ds(c * 16, 16)] = zeros16 + jnp.maximum(best, val_c)
            return ecur + 1

        lax.while_loop(lambda i0: i0 < T, run_body, i32(0))

        @plsc.parallel_loop(0, T // 16, unroll=4)
        def _sel(k):
            tv = tau_v[pl.ds(k * 16, 16)]
            tc = jnp.clip(tv, 0, T - 1)
            vs = plsc.load_gather(vid_v, [tc])
            tau_v[pl.ds(k * 16, 16)] = jnp.where(tv >= 0, vs, Dv)
        pltpu.sync_copy(tau_v.at[pl.ds(0, T)], vsel_hbm.at[row])


def _emb_body(M, Dv, T, d_out, vsel_ref, vemb_ref, out_ref):
    i32 = jnp.int32
    iota_dv_t = lax.broadcasted_iota(i32, (Dv, T), 0)
    for m in range(M):
        vs = vsel_ref[0, m, :]
        ohT = (vs[None, :] == iota_dv_t).astype(jnp.float32)
        res = lax.dot_general(
            ohT, vemb_ref[m], (((0,), (0,)), ((), ())),
            precision=lax.Precision.HIGHEST,
            preferred_element_type=jnp.float32)
        out_ref[0, :, m * d_out:(m + 1) * d_out] = res


def kernel(qk_logits_btmd, v_logits_btmd, v_emb_mk_d):
    B, T, M, Dqk = qk_logits_btmd.shape
    Dv = v_logits_btmd.shape[-1]
    d_out = v_emb_mk_d.shape[-1]
    N = B * M
    i32 = jnp.int32

    qk_dnt = qk_logits_btmd.transpose(3, 0, 2, 1).reshape(Dqk, N, T)
    v_dnt = v_logits_btmd.transpose(3, 0, 2, 1).reshape(Dv, N, T)

    q_nt, vid_nt, e_nt = pl.pallas_call(
        functools.partial(_pre_body, Dqk, Dv, N, T),
        out_shape=[jax.ShapeDtypeStruct((N, T), i32)] * 3,
    )(qk_dnt, v_dnt)

    mesh = plsc.VectorSubcoreMesh(core_axis_name="c", subcore_axis_name="s")
    sc_params = pltpu.CompilerParams(needs_layout_passes=False)
    vsel_nt = pl.kernel(
        functools.partial(_dp_body, N, T, Dv),
        out_type=jax.ShapeDtypeStruct((N, T), i32),
        mesh=mesh,
        scratch_types=[
            pltpu.VMEM((T + 16,), i32),
            pltpu.VMEM((T + 16,), i32),
            pltpu.VMEM((T + 16,), i32),
            pltpu.VMEM((T,), i32),
            pltpu.VMEM((16 * 16 + 16,), i32),
        ],
        compiler_params=sc_params,
    )(q_nt, e_nt, vid_nt)

    vsel_bmt = vsel_nt.reshape(B, M, T)
    out_flat = pl.pallas_call(
        functools.partial(_emb_body, M, Dv, T, d_out),
        grid=(B,),
        in_specs=[
            pl.BlockSpec((1, M, T), lambda b: (b, 0, 0)),
            pl.BlockSpec((M, Dv, d_out), lambda b: (0, 0, 0)),
        ],
        out_specs=pl.BlockSpec((1, T, M * d_out), lambda b: (b, 0, 0)),
        out_shape=jax.ShapeDtypeStruct((B, T, M * d_out), jnp.float32),
    )(vsel_bmt, v_emb_mk_d)

    return out_flat.reshape(B, T, M, d_out)


# speedup vs baseline: 87.7201x; 1.0726x over previous
import functools

import jax
import jax.numpy as jnp
from jax import lax
from jax.experimental import pallas as pl
from jax.experimental.pallas import tpu as pltpu
from jax.experimental.pallas import tpu_sc as plsc


def _pre_body(Dqk, Dv, N, T, PK, qk_ref, v_ref, qe_out, vid_out):
    i32 = jnp.int32
    best = qk_ref[0]
    bidx = jnp.zeros((N, T), i32)
    for d in range(1, Dqk):
        x = qk_ref[d]
        gt = x > best
        best = jnp.where(gt, x, best)
        bidx = jnp.where(gt, d, bidx)
    q = bidx

    vbest = v_ref[0]
    vidx = jnp.zeros((N, T), i32)
    for d in range(1, Dv):
        x = v_ref[d]
        gt = x > vbest
        vbest = jnp.where(gt, x, vbest)
        vidx = jnp.where(gt, d, vidx)
    vid_out[...] = vidx

    qnext = jnp.concatenate([q[:, 1:], jnp.full((N, 1), -1, i32)], axis=1)
    is_end = q != qnext
    iota_t = lax.broadcasted_iota(i32, (N, T), 1)
    e = jnp.where(is_end, iota_t, T)
    sh = 1
    while sh < T:
        shifted = jnp.concatenate(
            [e[:, sh:], jnp.full((N, sh), T, i32)], axis=1)
        e = jnp.minimum(e, shifted)
        sh *= 2
    qe_out[...] = e * PK + q


def _sload(ref, i):
    return ref[pl.ds(i, 16)][0]


def _dp_body(N, T, Dv, PK, qe_hbm, vid_hbm, vsel_hbm,
             qe_v, tau_v, vid_v, bk_v):
    i32 = jnp.int32
    cid = lax.axis_index("c")
    sid = lax.axis_index("s")
    wid = sid * 2 + cid

    @pl.when(wid < N)
    def _():
        row = wid
        pltpu.sync_copy(qe_hbm.at[row], qe_v.at[pl.ds(0, T)])
        pltpu.sync_copy(vid_hbm.at[row], vid_v.at[pl.ds(0, T)])
        iota = lax.iota(i32, 16)
        neg1 = jnp.full((16,), -1, i32)
        zeros16 = jnp.zeros((16,), i32)

        @plsc.parallel_loop(0, 17)
        def _init(k):
            bk_v[pl.ds(k * 16, 16)] = neg1

        def run_body(i0):
            qe = _sload(qe_v, i0)
            c = jnp.bitwise_and(qe, PK - 1)
            ecur = lax.shift_right_logical(qe, 4)
            best = _sload(bk_v, c * 16)
            L = ecur - i0 + 1
            has = best >= 0
            w0 = jnp.where(has, jnp.bitwise_and(best, 4095), 0)
            s0 = lax.shift_right_arithmetic(best, 12)
            e0 = lax.shift_right_logical(_sload(qe_v, w0), 4)

            tau_v[pl.ds(i0, 16)] = jnp.where(
                has, jnp.minimum(w0 + iota, e0), neg1)

            @pl.when(L > 16)
            def _():
                def wtau(k, carry):
                    tau_v[pl.ds(i0 + k * 16, 16)] = jnp.where(
                        has, jnp.minimum(w0 + k * 16 + iota, e0), neg1)
                    return carry
                lax.fori_loop(1, (L + 15) // 16, wtau, 0)

            wlen = jnp.minimum(L, e0 - w0 + 1)
            ns = jnp.minimum(wlen, e0 - w0)

            bcond = has & (wlen > e0 - w0) & (e0 + 1 < T)
            p = jnp.where(bcond, e0 + 1, 0)
            cb = jnp.bitwise_and(_sload(qe_v, p), PK - 1)
            old = _sload(bk_v, cb * 16)
            keyb = jnp.where(bcond, (s0 + wlen) * 4096 + e0 + 1, -1)
            bk_v[pl.ds(cb * 16, 16)] = zeros16 + jnp.maximum(old, keyb)

            val_c = jnp.where(has & (ns > 0), (s0 + ns) * 4096 + w0 + ns,
                              4096 + ecur)
            bk_v[pl.ds(c * 16, 16)] = zeros16 + jnp.maximum(best, val_c)
            return ecur + 1

        lax.while_loop(lambda i0: i0 < T, run_body, i32(0))

        @plsc.parallel_loop(0, T // 16, unroll=4)
        def _sel(k):
            tv = tau_v[pl.ds(k * 16, 16)]
            tc = jnp.clip(tv, 0, T - 1)
            vs = plsc.load_gather(vid_v, [tc])
            tau_v[pl.ds(k * 16, 16)] = jnp.where(tv >= 0, vs, Dv)
        pltpu.sync_copy(tau_v.at[pl.ds(0, T)], vsel_hbm.at[row])


def _emb_body(M, Dv, T, d_out, vsel_ref, vemb_ref, out_ref):
    i32 = jnp.int32
    iota_dv_t = lax.broadcasted_iota(i32, (Dv, T), 0)
    for m in range(M):
        vs = vsel_ref[0, m, :]
        ohT = (vs[None, :] == iota_dv_t).astype(jnp.float32)
        res = lax.dot_general(
            ohT, vemb_ref[m], (((0,), (0,)), ((), ())),
            precision=lax.Precision.HIGHEST,
            preferred_element_type=jnp.float32)
        out_ref[0, :, m * d_out:(m + 1) * d_out] = res


def kernel(qk_logits_btmd, v_logits_btmd, v_emb_mk_d):
    B, T, M, Dqk = qk_logits_btmd.shape
    Dv = v_logits_btmd.shape[-1]
    d_out = v_emb_mk_d.shape[-1]
    N = B * M
    PK = 16
    i32 = jnp.int32

    qk_dnt = qk_logits_btmd.transpose(3, 0, 2, 1).reshape(Dqk, N, T)
    v_dnt = v_logits_btmd.transpose(3, 0, 2, 1).reshape(Dv, N, T)

    qe_nt, vid_nt = pl.pallas_call(
        functools.partial(_pre_body, Dqk, Dv, N, T, PK),
        out_shape=[jax.ShapeDtypeStruct((N, T), i32)] * 2,
    )(qk_dnt, v_dnt)

    mesh = plsc.VectorSubcoreMesh(core_axis_name="c", subcore_axis_name="s")
    sc_params = pltpu.CompilerParams(needs_layout_passes=False)
    vsel_nt = pl.kernel(
        functools.partial(_dp_body, N, T, Dv, PK),
        out_type=jax.ShapeDtypeStruct((N, T), i32),
        mesh=mesh,
        scratch_types=[
            pltpu.VMEM((T + 16,), i32),
            pltpu.VMEM((T + 16,), i32),
            pltpu.VMEM((T,), i32),
            pltpu.VMEM((16 * 16 + 16,), i32),
        ],
        compiler_params=sc_params,
    )(qe_nt, vid_nt)

    vsel_bmt = vsel_nt.reshape(B, M, T)
    out_flat = pl.pallas_call(
        functools.partial(_emb_body, M, Dv, T, d_out),
        grid=(B,),
        in_specs=[
            pl.BlockSpec((1, M, T), lambda b: (b, 0, 0)),
            pl.BlockSpec((M, Dv, d_out), lambda b: (0, 0, 0)),
        ],
        out_specs=pl.BlockSpec((1, T, M * d_out), lambda b: (b, 0, 0)),
        out_shape=jax.ShapeDtypeStruct((B, T, M * d_out), jnp.float32),
    )(vsel_bmt, v_emb_mk_d)

    return out_flat.reshape(B, T, M, d_out)


# speedup vs baseline: 138.7199x; 1.5814x over previous
import functools

import jax
import jax.numpy as jnp
from jax import lax
from jax.experimental import pallas as pl
from jax.experimental.pallas import tpu as pltpu
from jax.experimental.pallas import tpu_sc as plsc


def _pre_body(Dqk, Dv, N, T, PK, qk_ref, v_ref, qe_out, vid_out):
    i32 = jnp.int32
    best = qk_ref[0]
    bidx = jnp.zeros((N, T), i32)
    for d in range(1, Dqk):
        x = qk_ref[d]
        gt = x > best
        best = jnp.where(gt, x, best)
        bidx = jnp.where(gt, d, bidx)
    q = bidx

    vbest = v_ref[0]
    vidx = jnp.zeros((N, T), i32)
    for d in range(1, Dv):
        x = v_ref[d]
        gt = x > vbest
        vbest = jnp.where(gt, x, vbest)
        vidx = jnp.where(gt, d, vidx)
    vid_out[...] = vidx

    qnext = jnp.concatenate([q[:, 1:], jnp.full((N, 1), -1, i32)], axis=1)
    is_end = q != qnext
    iota_t = lax.broadcasted_iota(i32, (N, T), 1)
    e = jnp.where(is_end, iota_t, T)
    sh = 1
    while sh < T:
        shifted = jnp.concatenate(
            [e[:, sh:], jnp.full((N, sh), T, i32)], axis=1)
        e = jnp.minimum(e, shifted)
        sh *= 2
    qe_out[...] = e * PK + q


_GD = lax.GatherDimensionNumbers(
    offset_dims=(), collapsed_slice_dims=(0,), start_index_map=(0,))


def _lanes(x, idx):
    return lax.gather(x, idx[:, None], _GD, (1,),
                      mode=lax.GatherScatterMode.PROMISE_IN_BOUNDS)


def _dp_body(N, T, Dv, PK, qe_hbm, vid_hbm, vsel_hbm,
             qe_v, tau_v, vid_v):
    i32 = jnp.int32
    cid = lax.axis_index("c")
    sid = lax.axis_index("s")
    wid = sid * 2 + cid

    @pl.when(wid < N)
    def _():
        row = wid
        pltpu.sync_copy(qe_hbm.at[row], qe_v.at[pl.ds(0, T)])
        pltpu.sync_copy(vid_hbm.at[row], vid_v.at[pl.ds(0, T)])
        iota = lax.iota(i32, 16)
        neg1 = jnp.full((16,), -1, i32)
        zero16 = jnp.zeros((16,), i32)
        lane0 = iota == 0

        def run_body(carry):
            i0s, i0v, bk_k, bk_e = carry
            qev = qe_v[pl.ds(i0s, 16)]
            qe0 = _lanes(qev, zero16)
            cv = jnp.bitwise_and(qe0, PK - 1)
            ecv = lax.shift_right_logical(qe0, 4)
            ecs = lax.shift_right_logical(qev[0], 4)
            bestv = _lanes(bk_k, cv)
            ebv = _lanes(bk_e, cv)
            hasv = bestv >= 0
            w0v = jnp.where(hasv, jnp.bitwise_and(bestv, 4095), 0)
            s0v = lax.shift_right_arithmetic(bestv, 12)
            e0v = ebv
            Lv = ecv - i0v + 1

            tau_v[pl.ds(i0s, 16)] = jnp.where(
                hasv, jnp.minimum(w0v + iota, e0v), neg1)

            @pl.when(ecs - i0s >= 16)
            def _():
                def wtau(k, carry2):
                    tau_v[pl.ds(i0s + k * 16, 16)] = jnp.where(
                        hasv, jnp.minimum(w0v + k * 16 + iota, e0v), neg1)
                    return carry2
                lax.fori_loop(1, (ecs - i0s + 16) // 16, wtau, 0)

            wlenv = jnp.minimum(Lv, e0v - w0v + 1)
            nsv = jnp.minimum(wlenv, e0v - w0v)

            bcondv = hasv & (wlenv > e0v - w0v) & (e0v + 1 < T)
            pv = jnp.where(bcondv, e0v + 1, 0)
            qepv = plsc.load_gather(qe_v, [pv], mask=lane0)
            qep0 = _lanes(qepv, zero16)
            cbv = jnp.bitwise_and(qep0, PK - 1)
            oldk = _lanes(bk_k, cbv)
            olde = _lanes(bk_e, cbv)
            keybv = jnp.where(bcondv, (s0v + wlenv) * 4096 + e0v + 1, -1)
            newkb = jnp.maximum(oldk, keybv)
            neweb = jnp.where(keybv > oldk,
                              lax.shift_right_logical(qep0, 4), olde)
            isb = iota == cbv
            bk_k = jnp.where(isb, newkb, bk_k)
            bk_e = jnp.where(isb, neweb, bk_e)

            own = hasv & (nsv > 0)
            valc = jnp.where(own, (s0v + nsv) * 4096 + w0v + nsv,
                             4096 + ecv)
            eown = jnp.where(own, e0v, ecv)
            newkc = jnp.maximum(bestv, valc)
            newec = jnp.where(valc > bestv, eown, ebv)
            isc = iota == cv
            bk_k = jnp.where(isc, newkc, bk_k)
            bk_e = jnp.where(isc, newec, bk_e)
            return (ecs + 1, ecv + 1, bk_k, bk_e)

        lax.while_loop(lambda c: c[0] < T, run_body,
                       (i32(0), zero16, neg1, zero16))

        @plsc.parallel_loop(0, T // 16, unroll=4)
        def _sel(k):
            tv = tau_v[pl.ds(k * 16, 16)]
            tc = jnp.clip(tv, 0, T - 1)
            vs = plsc.load_gather(vid_v, [tc])
            tau_v[pl.ds(k * 16, 16)] = jnp.where(tv >= 0, vs, Dv)
        pltpu.sync_copy(tau_v.at[pl.ds(0, T)], vsel_hbm.at[row])


def _emb_body(M, Dv, T, d_out, vsel_ref, vemb_ref, out_ref):
    i32 = jnp.int32
    iota_dv_t = lax.broadcasted_iota(i32, (Dv, T), 0)
    for m in range(M):
        vs = vsel_ref[0, m, :]
        ohT = (vs[None, :] == iota_dv_t).astype(jnp.float32)
        res = lax.dot_general(
            ohT, vemb_ref[m], (((0,), (0,)), ((), ())),
            precision=lax.Precision.HIGHEST,
            preferred_element_type=jnp.float32)
        out_ref[0, :, m * d_out:(m + 1) * d_out] = res


def kernel(qk_logits_btmd, v_logits_btmd, v_emb_mk_d):
    B, T, M, Dqk = qk_logits_btmd.shape
    Dv = v_logits_btmd.shape[-1]
    d_out = v_emb_mk_d.shape[-1]
    N = B * M
    PK = 16
    i32 = jnp.int32

    qk_dnt = qk_logits_btmd.transpose(3, 0, 2, 1).reshape(Dqk, N, T)
    v_dnt = v_logits_btmd.transpose(3, 0, 2, 1).reshape(Dv, N, T)

    qe_nt, vid_nt = pl.pallas_call(
        functools.partial(_pre_body, Dqk, Dv, N, T, PK),
        out_shape=[jax.ShapeDtypeStruct((N, T), i32)] * 2,
    )(qk_dnt, v_dnt)

    mesh = plsc.VectorSubcoreMesh(core_axis_name="c", subcore_axis_name="s")
    sc_params = pltpu.CompilerParams(needs_layout_passes=False)
    vsel_nt = pl.kernel(
        functools.partial(_dp_body, N, T, Dv, PK),
        out_type=jax.ShapeDtypeStruct((N, T), i32),
        mesh=mesh,
        scratch_types=[
            pltpu.VMEM((T + 16,), i32),
            pltpu.VMEM((T + 16,), i32),
            pltpu.VMEM((T,), i32),
        ],
        compiler_params=sc_params,
    )(qe_nt, vid_nt)

    vsel_bmt = vsel_nt.reshape(B, M, T)
    out_flat = pl.pallas_call(
        functools.partial(_emb_body, M, Dv, T, d_out),
        grid=(B,),
        in_specs=[
            pl.BlockSpec((1, M, T), lambda b: (b, 0, 0)),
            pl.BlockSpec((M, Dv, d_out), lambda b: (0, 0, 0)),
        ],
        out_specs=pl.BlockSpec((1, T, M * d_out), lambda b: (b, 0, 0)),
        out_shape=jax.ShapeDtypeStruct((B, T, M * d_out), jnp.float32),
    )(vsel_bmt, v_emb_mk_d)

    return out_flat.reshape(B, T, M, d_out)


# speedup vs baseline: 165.7946x; 1.1952x over previous
import functools

import jax
import jax.numpy as jnp
from jax import lax
from jax.experimental import pallas as pl
from jax.experimental.pallas import tpu as pltpu
from jax.experimental.pallas import tpu_sc as plsc


def _pre_body(Dqk, Dv, N, T, PK, qk_ref, v_ref, qe_out, vid_out):
    i32 = jnp.int32
    best = qk_ref[0]
    bidx = jnp.zeros((N, T), i32)
    for d in range(1, Dqk):
        x = qk_ref[d]
        gt = x > best
        best = jnp.where(gt, x, best)
        bidx = jnp.where(gt, d, bidx)
    q = bidx

    vbest = v_ref[0]
    vidx = jnp.zeros((N, T), i32)
    for d in range(1, Dv):
        x = v_ref[d]
        gt = x > vbest
        vbest = jnp.where(gt, x, vbest)
        vidx = jnp.where(gt, d, vidx)
    vid_out[...] = vidx

    qnext = jnp.concatenate([q[:, 1:], jnp.full((N, 1), -1, i32)], axis=1)
    is_end = q != qnext
    iota_t = lax.broadcasted_iota(i32, (N, T), 1)
    e = jnp.where(is_end, iota_t, T)
    sh = 1
    while sh < T:
        shifted = jnp.concatenate(
            [e[:, sh:], jnp.full((N, sh), T, i32)], axis=1)
        e = jnp.minimum(e, shifted)
        sh *= 2
    qe_out[...] = e * PK + q


_GD = lax.GatherDimensionNumbers(
    offset_dims=(), collapsed_slice_dims=(0,), start_index_map=(0,))


def _lanes(x, idx):
    return lax.gather(x, idx[:, None], _GD, (1,),
                      mode=lax.GatherScatterMode.PROMISE_IN_BOUNDS)


def _dp_body(N, T, Dv, PK, qe_hbm, vid_hbm, vsel_hbm,
             qe_v, tau_v, vid_v):
    i32 = jnp.int32
    cid = lax.axis_index("c")
    sid = lax.axis_index("s")
    wid = sid * 2 + cid

    @pl.when(wid < N)
    def _():
        row = wid
        pltpu.sync_copy(qe_hbm.at[row], qe_v.at[pl.ds(0, T)])
        pltpu.sync_copy(vid_hbm.at[row], vid_v.at[pl.ds(0, T)])
        iota = lax.iota(i32, 16)
        neg1 = jnp.full((16,), -1, i32)
        zero16 = jnp.zeros((16,), i32)
        lane0 = iota == 0
        qe_v[pl.ds(T, 16)] = zero16

        def run_body(carry):
            i0s, qes, i0v, pendv, bk_k, bk_e, bk_p = carry
            qe0 = zero16 + qes
            cv = jnp.bitwise_and(qe0, PK - 1)
            ecv = lax.shift_right_logical(qe0, 4)
            ecs = lax.shift_right_logical(qes, 4)
            i0n = ecs + 1
            qesn = qe_v[pl.ds(i0n, 16)][0]

            isp = iota == pendv
            bk_p = jnp.where(isp, qe0, bk_p)

            bestv = _lanes(bk_k, cv)
            ebv = _lanes(bk_e, cv)
            bpv = _lanes(bk_p, cv)
            hasv = bestv >= 0
            w0v = jnp.where(hasv, jnp.bitwise_and(bestv, 4095), 0)
            s0v = lax.shift_right_arithmetic(bestv, 12)
            e0v = ebv

            tau_v[pl.ds(i0s, 16)] = jnp.where(
                hasv, jnp.minimum(w0v + iota, e0v), neg1)

            @pl.when(ecs - i0s >= 16)
            def _():
                def wtau(k, carry2):
                    tau_v[pl.ds(i0s + k * 16, 16)] = jnp.where(
                        hasv, jnp.minimum(w0v + k * 16 + iota, e0v), neg1)
                    return carry2
                lax.fori_loop(1, (ecs - i0s + 16) // 16, wtau, 0)

            Lv = ecv - i0v + 1
            wlenv = jnp.minimum(Lv, e0v - w0v + 1)
            nsv = jnp.minimum(wlenv, e0v - w0v)

            bcondv = hasv & (wlenv > e0v - w0v) & (e0v + 1 < T)
            cbv = jnp.bitwise_and(bpv, PK - 1)
            epv = lax.shift_right_logical(bpv, 4)
            npv = plsc.load_gather(qe_v, [epv + 1], mask=lane0)
            np0 = _lanes(npv, zero16)
            oldk = _lanes(bk_k, cbv)
            olde = _lanes(bk_e, cbv)
            oldp = _lanes(bk_p, cbv)
            keybv = jnp.where(bcondv, (s0v + wlenv) * 4096 + e0v + 1, -1)
            winb = keybv > oldk
            isb = iota == cbv
            bk_k = jnp.where(isb, jnp.maximum(oldk, keybv), bk_k)
            bk_e = jnp.where(isb, jnp.where(winb, epv, olde), bk_e)
            bk_p = jnp.where(isb, jnp.where(winb, np0, oldp), bk_p)

            own = hasv & (nsv > 0)
            valc = jnp.where(own, (s0v + nsv) * 4096 + w0v + nsv,
                             4096 + ecv)
            winc = valc > bestv
            isc = iota == cv
            bk_k = jnp.where(isc, jnp.maximum(bestv, valc), bk_k)
            bk_e = jnp.where(isc, jnp.where(winc, jnp.where(own, e0v, ecv),
                                            ebv), bk_e)
            pendv = jnp.where(winc & ~own, cv, neg1)
            return (i0n, qesn, ecv + 1, pendv, bk_k, bk_e, bk_p)

        qes0 = qe_v[pl.ds(0, 16)][0]
        lax.while_loop(lambda c: c[0] < T, run_body,
                       (i32(0), qes0, zero16, neg1, neg1, zero16, zero16))

        @plsc.parallel_loop(0, T // 16, unroll=4)
        def _sel(k):
            tv = tau_v[pl.ds(k * 16, 16)]
            tc = jnp.clip(tv, 0, T - 1)
            vs = plsc.load_gather(vid_v, [tc])
            tau_v[pl.ds(k * 16, 16)] = jnp.where(tv >= 0, vs, Dv)
        pltpu.sync_copy(tau_v.at[pl.ds(0, T)], vsel_hbm.at[row])


def _emb_body(M, Dv, T, d_out, vsel_ref, vemb_ref, out_ref):
    i32 = jnp.int32
    iota_dv_t = lax.broadcasted_iota(i32, (Dv, T), 0)
    for m in range(M):
        vs = vsel_ref[0, m, :]
        ohT = (vs[None, :] == iota_dv_t).astype(jnp.float32)
        res = lax.dot_general(
            ohT, vemb_ref[m], (((0,), (0,)), ((), ())),
            precision=lax.Precision.HIGHEST,
            preferred_element_type=jnp.float32)
        out_ref[0, :, m * d_out:(m + 1) * d_out] = res


def kernel(qk_logits_btmd, v_logits_btmd, v_emb_mk_d):
    B, T, M, Dqk = qk_logits_btmd.shape
    Dv = v_logits_btmd.shape[-1]
    d_out = v_emb_mk_d.shape[-1]
    N = B * M
    PK = 16
    i32 = jnp.int32

    qk_dnt = qk_logits_btmd.transpose(3, 0, 2, 1).reshape(Dqk, N, T)
    v_dnt = v_logits_btmd.transpose(3, 0, 2, 1).reshape(Dv, N, T)

    qe_nt, vid_nt = pl.pallas_call(
        functools.partial(_pre_body, Dqk, Dv, N, T, PK),
        out_shape=[jax.ShapeDtypeStruct((N, T), i32)] * 2,
    )(qk_dnt, v_dnt)

    mesh = plsc.VectorSubcoreMesh(core_axis_name="c", subcore_axis_name="s")
    sc_params = pltpu.CompilerParams(needs_layout_passes=False)
    vsel_nt = pl.kernel(
        functools.partial(_dp_body, N, T, Dv, PK),
        out_type=jax.ShapeDtypeStruct((N, T), i32),
        mesh=mesh,
        scratch_types=[
            pltpu.VMEM((T + 32,), i32),
            pltpu.VMEM((T + 16,), i32),
            pltpu.VMEM((T,), i32),
        ],
        compiler_params=sc_params,
    )(qe_nt, vid_nt)

    vsel_bmt = vsel_nt.reshape(B, M, T)
    out_flat = pl.pallas_call(
        functools.partial(_emb_body, M, Dv, T, d_out),
        grid=(B,),
        in_specs=[
            pl.BlockSpec((1, M, T), lambda b: (b, 0, 0)),
            pl.BlockSpec((M, Dv, d_out), lambda b: (0, 0, 0)),
        ],
        out_specs=pl.BlockSpec((1, T, M * d_out), lambda b: (b, 0, 0)),
        out_shape=jax.ShapeDtypeStruct((B, T, M * d_out), jnp.float32),
    )(vsel_bmt, v_emb_mk_d)

    return out_flat.reshape(B, T, M, d_out)
